# Initial kernel scaffold; baseline (speedup 1.0000x reference)
import jax
import jax.numpy as jnp
from jax.experimental import pallas as pl


def kernel(idx, A_param):
    raise NotImplementedError("write your pallas kernel here")


# speedup vs baseline: 13.7828x; 13.7828x over previous
import jax
import jax.numpy as jnp
from jax.experimental import pallas as pl

_K = 32


def _body(a_ref, o_ref):
    v = jnp.maximum(a_ref[...], 0.0)
    vi = jax.lax.bitcast_convert_type(v, jnp.int32)
    rows = v.shape[0]

    def step(i, t):
        bit = jax.lax.shift_left(jnp.int32(1), jnp.int32(30) - i)
        cand = jnp.bitwise_or(t, bit)
        cnt = jnp.sum((vi >= cand).astype(jnp.int32), axis=1, keepdims=True)
        return jnp.where(cnt >= _K, cand, t)

    t = jax.lax.fori_loop(0, 31, step, jnp.zeros((rows, 1), jnp.int32))
    o_ref[...] = jnp.where(vi >= t, v, 0.0)


def kernel(idx, A_param):
    n, m = A_param.shape
    br = 200 if n % 200 == 0 else n
    return pl.pallas_call(
        _body,
        grid=(n // br,),
        in_specs=[pl.BlockSpec((br, m), lambda i: (i, 0))],
        out_specs=pl.BlockSpec((br, m), lambda i: (i, 0)),
        out_shape=jax.ShapeDtypeStruct((n, m), jnp.float32),
    )(A_param)
